# baseline (speedup 1.0000x reference)
import functools

import jax
import jax.numpy as jnp
from jax import lax
from jax.experimental import pallas as pl
from jax.experimental.pallas import tpu as pltpu
from jax.experimental.pallas import tpu_sc as plsc

_D = 64
_SCALE = 8.0
_LANES = 16
_VB = 128
_CHUNK = 128
_NBUF = 5


@functools.lru_cache(maxsize=None)
def _make_detile(vocab, num_cores, num_subcores):
    num_workers = num_cores * num_subcores
    n_full = vocab // _VB
    tail = vocab - n_full * _VB
    base_cnt = n_full // num_workers
    extra = n_full % num_workers
    mesh = plsc.VectorSubcoreMesh(core_axis_name="c", subcore_axis_name="s")

    @functools.partial(
        pl.kernel,
        mesh=mesh,
        out_type=jax.ShapeDtypeStruct((vocab * _D,), jnp.float32),
        scratch_types=[
            pltpu.VMEM((2, 8, 8, _VB), jnp.float32),
            pltpu.VMEM((2, _VB * _D), jnp.float32),
            pltpu.SemaphoreType.DMA((2,)),
            pltpu.SemaphoreType.DMA((2,)),
        ],
        compiler_params=pltpu.CompilerParams(
            use_tc_tiling_on_sc=True, needs_layout_passes=False
        ),
    )
    def ka(t3_hbm, tail_hbm, out_hbm, stage, rows, isem, osem):
        wid = lax.axis_index("s") * num_cores + lax.axis_index("c")
        start = wid * base_cnt + jnp.minimum(wid, extra)
        cnt = base_cnt + jnp.where(wid < extra, 1, 0)
        stop = start + cnt

        lane = lax.iota(jnp.int32, _LANES)
        c_rg = [lax.shift_right_logical(16 * q + lane, 3) for q in range(4)]
        c_r8 = [lax.bitwise_and(16 * q + lane, 7) for q in range(4)]

        def fire_in(j, si, width):
            for rg in range(8):
                pltpu.async_copy(
                    t3_hbm.at[rg, :, pl.ds(j * _VB, width)],
                    stage.at[si, rg, :, pl.ds(0, width)],
                    isem.at[si],
                )

        def wait_in(j, si, width):
            for rg in range(8):
                pltpu.make_async_copy(
                    t3_hbm.at[rg, :, pl.ds(j * _VB, width)],
                    stage.at[si, rg, :, pl.ds(0, width)],
                    isem.at[si],
                ).wait()

        def assemble(si, nrows):
            def row(r, carry):
                rv = jnp.full((_LANES,), r, jnp.int32)
                for q in range(4):
                    v = plsc.load_gather(stage.at[si], [c_rg[q], c_r8[q], rv])
                    rows[si, pl.ds(r * _D + 16 * q, _LANES)] = v
                return carry

            lax.fori_loop(0, nrows, row, None)

        def fire_out(j, si, nrows):
            pltpu.async_copy(
                rows.at[si, pl.ds(0, nrows * _D)],
                out_hbm.at[pl.ds(j * _VB * _D, nrows * _D)],
                osem.at[si],
            )

        def wait_out(si, nrows):
            pltpu.make_async_copy(
                rows.at[si, pl.ds(0, nrows * _D)],
                out_hbm.at[pl.ds(0, nrows * _D)],
                osem.at[si],
            ).wait()

        @pl.when(cnt > 0)
        def _prologue():
            fire_in(start, 0, _VB)

        def body(t, carry):
            j0 = start + 2 * t
            j1 = j0 + 1

            def half(j, si, jn):
                wait_in(j, si, _VB)

                @pl.when(jn < stop)
                def _():
                    fire_in(jn, 1 - si, _VB)

                @pl.when(t >= 1)
                def _():
                    wait_out(si, _VB)

                assemble(si, _VB)
                fire_out(j, si, _VB)

            @pl.when(j0 < stop)
            def _():
                half(j0, 0, j1)

            @pl.when(j1 < stop)
            def _():
                half(j1, 1, j0 + 2)

            return carry

        lax.fori_loop(0, (cnt + 1) // 2, body, None)

        @pl.when(cnt >= 1)
        def _():
            wait_out(0, _VB)

        @pl.when(cnt >= 2)
        def _():
            wait_out(1, _VB)

        if tail:
            @pl.when(wid == extra)
            def _tail():
                pltpu.sync_copy(tail_hbm, rows.at[0, pl.ds(0, tail * _D)])
                pltpu.sync_copy(
                    rows.at[0, pl.ds(0, tail * _D)],
                    out_hbm.at[pl.ds(n_full * _VB * _D, tail * _D)],
                )

    return ka


@functools.lru_cache(maxsize=None)
def _make_gather(vocab, batch, num_cores, num_subcores):
    num_workers = num_cores * num_subcores
    group = _CHUNK * _NBUF
    assert batch % (num_workers * 2 * group) == 0
    b_per_w = batch // num_workers
    n_groups = b_per_w // group
    mesh = plsc.VectorSubcoreMesh(core_axis_name="c", subcore_axis_name="s")

    @functools.partial(
        pl.kernel,
        mesh=mesh,
        out_type=jax.ShapeDtypeStruct((batch, _D), jnp.float32),
        scratch_types=[
            pltpu.VMEM((b_per_w,), jnp.int32),
            pltpu.VMEM((2, _NBUF, _CHUNK, _D), jnp.float32),
            pltpu.SemaphoreType.DMA((2,)),
            pltpu.SemaphoreType.DMA((2,)),
        ],
        compiler_params=pltpu.CompilerParams(use_tc_tiling_on_sc=False),
    )
    def k(idx_hbm, table_hbm, out_hbm, idx_v, rows_v, gsem, wsem):
        wid = lax.axis_index("s") * num_cores + lax.axis_index("c")
        base = wid * b_per_w
        pltpu.sync_copy(idx_hbm.at[pl.ds(base, b_per_w)], idx_v)

        def fire_gathers(t, si):
            for b in range(_NBUF):
                off = (t * _NBUF + b) * _CHUNK
                pltpu.async_copy(
                    table_hbm.at[idx_v.at[pl.ds(off, _CHUNK)]],
                    rows_v.at[si, b],
                    gsem.at[si],
                )

        def drain(sem_arr, si, hbm_side):
            for b in range(_NBUF):
                pltpu.make_async_copy(hbm_side, rows_v.at[si, b], sem_arr.at[si]).wait()

        fire_gathers(0, 0)
        for t in range(n_groups):
            si = t % 2
            ni = 1 - si
            if t + 1 < n_groups:
                if t >= 1:
                    drain(wsem, ni, out_hbm.at[pl.ds(base, _CHUNK)])
                fire_gathers(t + 1, ni)
            drain(gsem, si, table_hbm.at[idx_v.at[pl.ds(0, _CHUNK)]])

            def scale_row(r, carry):
                for b in range(_NBUF):
                    for q in range(_D // _LANES):
                        sl = pl.ds(q * _LANES, _LANES)
                        rows_v[si, b, r, sl] = rows_v[si, b, r, sl] * _SCALE
                return carry

            lax.fori_loop(0, _CHUNK, scale_row, None)

            for b in range(_NBUF):
                off = (t * _NBUF + b) * _CHUNK
                pltpu.async_copy(
                    rows_v.at[si, b],
                    out_hbm.at[pl.ds(base + off, _CHUNK)],
                    wsem.at[si],
                )
        drain(wsem, (n_groups - 1) % 2, out_hbm.at[pl.ds(base, _CHUNK)])

    return k


def kernel(x, embedding):
    b0, s = x.shape
    batch = b0 * s
    vocab = embedding.shape[0]
    info = plsc.get_sparse_core_info()
    t3 = embedding.T.reshape(8, _D // 8, vocab)
    n_full = vocab // _VB
    tail_rows = embedding[n_full * _VB :, :].reshape(-1)
    table_lin = _make_detile(vocab, info.num_cores, info.num_subcores)(
        t3, tail_rows
    )
    table_rm = table_lin.reshape(vocab, _D)
    xf = x.reshape(batch)
    out = _make_gather(vocab, batch, info.num_cores, info.num_subcores)(
        xf, table_rm
    )
    return out.reshape(b0, s, _D)


# speedup vs baseline: 1.1411x; 1.1411x over previous
import functools

import jax
import jax.numpy as jnp
from jax import lax
from jax.experimental import pallas as pl
from jax.experimental.pallas import tpu as pltpu
from jax.experimental.pallas import tpu_sc as plsc

_D = 64
_SCALE = 8.0
_LANES = 16
_VB = 128
_CHUNK = 128
_NBUF = 5


@functools.lru_cache(maxsize=None)
def _make_detile(vocab, num_cores, num_subcores):
    num_workers = num_cores * num_subcores
    n_full = vocab // _VB
    tail = vocab - n_full * _VB
    base_cnt = n_full // num_workers
    extra = n_full % num_workers
    mesh = plsc.VectorSubcoreMesh(core_axis_name="c", subcore_axis_name="s")

    @functools.partial(
        pl.kernel,
        mesh=mesh,
        out_type=jax.ShapeDtypeStruct((vocab * _D,), jnp.float32),
        scratch_types=[
            pltpu.VMEM((8, 8, _VB), jnp.float32),
            pltpu.VMEM((8, 8, _VB), jnp.float32),
            pltpu.VMEM((_VB * _D,), jnp.float32),
            pltpu.VMEM((_VB * _D,), jnp.float32),
            pltpu.SemaphoreType.DMA((2,)),
            pltpu.SemaphoreType.DMA((2,)),
        ],
        compiler_params=pltpu.CompilerParams(
            use_tc_tiling_on_sc=True, needs_layout_passes=False
        ),
    )
    def ka(t3_hbm, tail_hbm, out_hbm, stage0, stage1, rows0, rows1, isem, osem):
        stage = [stage0, stage1]
        rows = [rows0, rows1]
        wid = lax.axis_index("s") * num_cores + lax.axis_index("c")
        start = wid * base_cnt + jnp.minimum(wid, extra)
        cnt = base_cnt + jnp.where(wid < extra, 1, 0)
        stop = start + cnt

        lane = lax.iota(jnp.int32, _LANES)
        lane64 = lane * _D

        def fire_in(j, si, width):
            for rg in range(8):
                pltpu.async_copy(
                    t3_hbm.at[rg, :, pl.ds(j * _VB, width)],
                    stage[si].at[rg, :, pl.ds(0, width)],
                    isem.at[si],
                )

        def wait_in(j, si, width):
            for rg in range(8):
                pltpu.make_async_copy(
                    t3_hbm.at[rg, :, pl.ds(j * _VB, width)],
                    stage[si].at[rg, :, pl.ds(0, width)],
                    isem.at[si],
                ).wait()

        def assemble(si, nrows):
            del nrows

            def mbody(m, carry):
                i2 = lane + m * _LANES
                base_v = lane64 + m * (_LANES * _D)
                for rg in range(8):
                    i0 = jnp.full((_LANES,), rg, jnp.int32)
                    for r8 in range(8):
                        i1 = jnp.full((_LANES,), r8, jnp.int32)
                        v = plsc.load_gather(stage[si], [i0, i1, i2])
                        plsc.store_scatter(
                            rows[si], [base_v + (rg * 8 + r8)], v * _SCALE
                        )
                return carry

            lax.fori_loop(0, _VB // _LANES, mbody, None)

        def fire_out(j, si, nrows):
            pltpu.async_copy(
                rows[si].at[pl.ds(0, nrows * _D)],
                out_hbm.at[pl.ds(j * _VB * _D, nrows * _D)],
                osem.at[si],
            )

        def wait_out(si, nrows):
            pltpu.make_async_copy(
                rows[si].at[pl.ds(0, nrows * _D)],
                out_hbm.at[pl.ds(0, nrows * _D)],
                osem.at[si],
            ).wait()

        @pl.when(cnt > 0)
        def _prologue():
            fire_in(start, 0, _VB)

        def body(t, carry):
            j0 = start + 2 * t
            j1 = j0 + 1

            def half(j, si, jn):
                wait_in(j, si, _VB)

                @pl.when(jn < stop)
                def _():
                    fire_in(jn, 1 - si, _VB)

                @pl.when(t >= 1)
                def _():
                    wait_out(si, _VB)

                assemble(si, _VB)
                fire_out(j, si, _VB)

            @pl.when(j0 < stop)
            def _():
                half(j0, 0, j1)

            @pl.when(j1 < stop)
            def _():
                half(j1, 1, j0 + 2)

            return carry

        lax.fori_loop(0, (cnt + 1) // 2, body, None)

        @pl.when(cnt >= 1)
        def _():
            wait_out(0, _VB)

        @pl.when(cnt >= 2)
        def _():
            wait_out(1, _VB)

        if tail:
            @pl.when(wid == extra)
            def _tail():
                pltpu.sync_copy(tail_hbm, rows[0].at[pl.ds(0, tail * _D)])
                pltpu.sync_copy(
                    rows[0].at[pl.ds(0, tail * _D)],
                    out_hbm.at[pl.ds(n_full * _VB * _D, tail * _D)],
                )

    return ka


@functools.lru_cache(maxsize=None)
def _make_gather(vocab, batch, num_cores, num_subcores):
    num_workers = num_cores * num_subcores
    group = _CHUNK * _NBUF
    assert batch % (num_workers * 2 * group) == 0
    b_per_w = batch // num_workers
    n_groups = b_per_w // group
    mesh = plsc.VectorSubcoreMesh(core_axis_name="c", subcore_axis_name="s")

    @functools.partial(
        pl.kernel,
        mesh=mesh,
        out_type=jax.ShapeDtypeStruct((batch, _D), jnp.float32),
        scratch_types=[
            pltpu.VMEM((b_per_w,), jnp.int32),
            pltpu.VMEM((2, _NBUF, _CHUNK, _D), jnp.float32),
            pltpu.SemaphoreType.DMA((2,)),
            pltpu.SemaphoreType.DMA((2,)),
        ],
        compiler_params=pltpu.CompilerParams(use_tc_tiling_on_sc=False),
    )
    def k(idx_hbm, table_hbm, out_hbm, idx_v, rows_v, gsem, wsem):
        wid = lax.axis_index("s") * num_cores + lax.axis_index("c")
        base = wid * b_per_w
        pltpu.sync_copy(idx_hbm.at[pl.ds(base, b_per_w)], idx_v)

        def fire_gathers(t, si):
            for b in range(_NBUF):
                off = (t * _NBUF + b) * _CHUNK
                pltpu.async_copy(
                    table_hbm.at[idx_v.at[pl.ds(off, _CHUNK)]],
                    rows_v.at[si, b],
                    gsem.at[si],
                )

        def drain(sem_arr, si, hbm_side):
            for b in range(_NBUF):
                pltpu.make_async_copy(hbm_side, rows_v.at[si, b], sem_arr.at[si]).wait()

        fire_gathers(0, 0)
        for t in range(n_groups):
            si = t % 2
            ni = 1 - si
            if t + 1 < n_groups:
                if t >= 1:
                    drain(wsem, ni, out_hbm.at[pl.ds(base, _CHUNK)])
                fire_gathers(t + 1, ni)
            drain(gsem, si, table_hbm.at[idx_v.at[pl.ds(0, _CHUNK)]])

            for b in range(_NBUF):
                off = (t * _NBUF + b) * _CHUNK
                pltpu.async_copy(
                    rows_v.at[si, b],
                    out_hbm.at[pl.ds(base + off, _CHUNK)],
                    wsem.at[si],
                )
        drain(wsem, (n_groups - 1) % 2, out_hbm.at[pl.ds(base, _CHUNK)])

    return k


def kernel(x, embedding):
    b0, s = x.shape
    batch = b0 * s
    vocab = embedding.shape[0]
    info = plsc.get_sparse_core_info()
    t3 = embedding.T.reshape(8, _D // 8, vocab)
    n_full = vocab // _VB
    tail_rows = (embedding[n_full * _VB :, :] * _SCALE).reshape(-1)
    table_lin = _make_detile(vocab, info.num_cores, info.num_subcores)(
        t3, tail_rows
    )
    table_rm = table_lin.reshape(vocab, _D)
    xf = x.reshape(batch)
    out = _make_gather(vocab, batch, info.num_cores, info.num_subcores)(
        xf, table_rm
    )
    return out.reshape(b0, s, _D)


# speedup vs baseline: 1.6386x; 1.4360x over previous
import functools

import jax
import jax.numpy as jnp
from jax import lax
from jax.experimental import pallas as pl
from jax.experimental.pallas import tpu as pltpu
from jax.experimental.pallas import tpu_sc as plsc

_D = 64
_SCALE = 8.0
_LANES = 16
_VB = 128
_CHUNK = 128
_NBUF = 5


@functools.lru_cache(maxsize=None)
def _make_detile(vocab, num_cores, num_subcores):
    num_workers = num_cores * num_subcores
    n_full = vocab // _VB
    tail = vocab - n_full * _VB
    base_cnt = n_full // num_workers
    extra = n_full % num_workers
    mesh = plsc.VectorSubcoreMesh(core_axis_name="c", subcore_axis_name="s")

    @functools.partial(
        pl.kernel,
        mesh=mesh,
        out_type=jax.ShapeDtypeStruct((vocab * _D,), jnp.float32),
        scratch_types=[
            pltpu.VMEM((8, 8, _VB), jnp.float32),
            pltpu.VMEM((8, 8, _VB), jnp.float32),
            pltpu.VMEM((_VB * _D,), jnp.float32),
            pltpu.VMEM((_VB * _D,), jnp.float32),
            pltpu.SemaphoreType.DMA((2,)),
            pltpu.SemaphoreType.DMA((2,)),
        ],
        compiler_params=pltpu.CompilerParams(
            use_tc_tiling_on_sc=True, needs_layout_passes=False
        ),
    )
    def ka(t3_hbm, tail_hbm, out_hbm, stage0, stage1, rows0, rows1, isem, osem):
        stage = [stage0, stage1]
        rows = [rows0, rows1]
        wid = lax.axis_index("s") * num_cores + lax.axis_index("c")
        start = wid * base_cnt + jnp.minimum(wid, extra)
        cnt = base_cnt + jnp.where(wid < extra, 1, 0)
        stop = start + cnt

        lane = lax.iota(jnp.int32, _LANES)
        lane64 = lane * _D

        def fire_in(j, si, width):
            for rg in range(8):
                pltpu.async_copy(
                    t3_hbm.at[rg, :, pl.ds(j * _VB, width)],
                    stage[si].at[rg, :, pl.ds(0, width)],
                    isem.at[si],
                )

        def wait_in(j, si, width):
            for rg in range(8):
                pltpu.make_async_copy(
                    t3_hbm.at[rg, :, pl.ds(j * _VB, width)],
                    stage[si].at[rg, :, pl.ds(0, width)],
                    isem.at[si],
                ).wait()

        def assemble(si, nrows):
            del nrows

            @plsc.parallel_loop(0, (_VB // _LANES) * _D, unroll=8)
            def _body(t):
                d = lax.bitwise_and(t, _D - 1)
                m = lax.shift_right_logical(t, 6)
                i0 = jnp.full((_LANES,), lax.shift_right_logical(d, 3), jnp.int32)
                i1 = jnp.full((_LANES,), lax.bitwise_and(d, 7), jnp.int32)
                i2 = lane + m * _LANES
                widx = lane64 + (m * (_LANES * _D) + d)
                v = plsc.load_gather(stage[si], [i0, i1, i2])
                plsc.store_scatter(rows[si], [widx], v * _SCALE)

        def fire_out(j, si, nrows):
            pltpu.async_copy(
                rows[si].at[pl.ds(0, nrows * _D)],
                out_hbm.at[pl.ds(j * _VB * _D, nrows * _D)],
                osem.at[si],
            )

        def wait_out(si, nrows):
            pltpu.make_async_copy(
                rows[si].at[pl.ds(0, nrows * _D)],
                out_hbm.at[pl.ds(0, nrows * _D)],
                osem.at[si],
            ).wait()

        @pl.when(cnt > 0)
        def _prologue():
            fire_in(start, 0, _VB)

        def body(t, carry):
            j0 = start + 2 * t
            j1 = j0 + 1

            def half(j, si, jn):
                wait_in(j, si, _VB)

                @pl.when(jn < stop)
                def _():
                    fire_in(jn, 1 - si, _VB)

                @pl.when(t >= 1)
                def _():
                    wait_out(si, _VB)

                assemble(si, _VB)
                fire_out(j, si, _VB)

            @pl.when(j0 < stop)
            def _():
                half(j0, 0, j1)

            @pl.when(j1 < stop)
            def _():
                half(j1, 1, j0 + 2)

            return carry

        lax.fori_loop(0, (cnt + 1) // 2, body, None)

        @pl.when(cnt >= 1)
        def _():
            wait_out(0, _VB)

        @pl.when(cnt >= 2)
        def _():
            wait_out(1, _VB)

        if tail:
            @pl.when(wid == extra)
            def _tail():
                pltpu.sync_copy(tail_hbm, rows[0].at[pl.ds(0, tail * _D)])
                pltpu.sync_copy(
                    rows[0].at[pl.ds(0, tail * _D)],
                    out_hbm.at[pl.ds(n_full * _VB * _D, tail * _D)],
                )

    return ka


@functools.lru_cache(maxsize=None)
def _make_gather(vocab, batch, num_cores, num_subcores):
    num_workers = num_cores * num_subcores
    group = _CHUNK * _NBUF
    assert batch % (num_workers * 2 * group) == 0
    b_per_w = batch // num_workers
    n_groups = b_per_w // group
    mesh = plsc.VectorSubcoreMesh(core_axis_name="c", subcore_axis_name="s")

    @functools.partial(
        pl.kernel,
        mesh=mesh,
        out_type=jax.ShapeDtypeStruct((batch, _D), jnp.float32),
        scratch_types=[
            pltpu.VMEM((b_per_w,), jnp.int32),
            pltpu.VMEM((2, _NBUF, _CHUNK, _D), jnp.float32),
            pltpu.SemaphoreType.DMA((2,)),
            pltpu.SemaphoreType.DMA((2,)),
        ],
        compiler_params=pltpu.CompilerParams(use_tc_tiling_on_sc=False),
    )
    def k(idx_hbm, table_hbm, out_hbm, idx_v, rows_v, gsem, wsem):
        wid = lax.axis_index("s") * num_cores + lax.axis_index("c")
        base = wid * b_per_w
        pltpu.sync_copy(idx_hbm.at[pl.ds(base, b_per_w)], idx_v)

        def fire_gathers(t, si):
            for b in range(_NBUF):
                off = (t * _NBUF + b) * _CHUNK
                pltpu.async_copy(
                    table_hbm.at[idx_v.at[pl.ds(off, _CHUNK)]],
                    rows_v.at[si, b],
                    gsem.at[si],
                )

        def drain(sem_arr, si, hbm_side):
            for b in range(_NBUF):
                pltpu.make_async_copy(hbm_side, rows_v.at[si, b], sem_arr.at[si]).wait()

        fire_gathers(0, 0)
        for t in range(n_groups):
            si = t % 2
            ni = 1 - si
            if t + 1 < n_groups:
                if t >= 1:
                    drain(wsem, ni, out_hbm.at[pl.ds(base, _CHUNK)])
                fire_gathers(t + 1, ni)
            drain(gsem, si, table_hbm.at[idx_v.at[pl.ds(0, _CHUNK)]])

            for b in range(_NBUF):
                off = (t * _NBUF + b) * _CHUNK
                pltpu.async_copy(
                    rows_v.at[si, b],
                    out_hbm.at[pl.ds(base + off, _CHUNK)],
                    wsem.at[si],
                )
        drain(wsem, (n_groups - 1) % 2, out_hbm.at[pl.ds(base, _CHUNK)])

    return k


def kernel(x, embedding):
    b0, s = x.shape
    batch = b0 * s
    vocab = embedding.shape[0]
    info = plsc.get_sparse_core_info()
    t3 = embedding.T.reshape(8, _D // 8, vocab)
    n_full = vocab // _VB
    tail_rows = (embedding[n_full * _VB :, :] * _SCALE).reshape(-1)
    table_lin = _make_detile(vocab, info.num_cores, info.num_subcores)(
        t3, tail_rows
    )
    table_rm = table_lin.reshape(vocab, _D)
    xf = x.reshape(batch)
    out = _make_gather(vocab, batch, info.num_cores, info.num_subcores)(
        xf, table_rm
    )
    return out.reshape(b0, s, _D)


# speedup vs baseline: 2.2599x; 1.3792x over previous
import functools

import jax
import jax.numpy as jnp
from jax import lax
from jax.experimental import pallas as pl
from jax.experimental.pallas import tpu as pltpu
from jax.experimental.pallas import tpu_sc as plsc

_D = 64
_SCALE = 8.0
_LANES = 16
_CHUNK = 128
_NBUF = 5


@functools.lru_cache(maxsize=None)
def _make(vocab, batch, num_cores, num_subcores):
    num_workers = num_cores * num_subcores
    group = _CHUNK * _NBUF
    assert batch % (num_workers * 2 * group) == 0
    b_per_w = batch // num_workers
    n_groups = b_per_w // group
    mesh = plsc.VectorSubcoreMesh(core_axis_name="c", subcore_axis_name="s")

    @functools.partial(
        pl.kernel,
        mesh=mesh,
        out_type=jax.ShapeDtypeStruct((batch, _D), jnp.float32),
        scratch_types=[
            pltpu.VMEM((b_per_w,), jnp.int32),
            pltpu.VMEM((2, _NBUF, _CHUNK, _D), jnp.float32),
            pltpu.SemaphoreType.DMA((2,)),
            pltpu.SemaphoreType.DMA((2,)),
        ],
        compiler_params=pltpu.CompilerParams(use_tc_tiling_on_sc=False),
    )
    def k(idx_hbm, table_hbm, out_hbm, idx_v, rows_v, gsem, wsem):
        wid = lax.axis_index("s") * num_cores + lax.axis_index("c")
        base = wid * b_per_w
        pltpu.sync_copy(idx_hbm.at[pl.ds(base, b_per_w)], idx_v)

        def fire_gathers(t, si):
            for b in range(_NBUF):
                off = (t * _NBUF + b) * _CHUNK
                pltpu.async_copy(
                    table_hbm.at[idx_v.at[pl.ds(off, _CHUNK)]],
                    rows_v.at[si, b],
                    gsem.at[si],
                )

        def drain(sem_arr, si, hbm_side):
            for b in range(_NBUF):
                pltpu.make_async_copy(hbm_side, rows_v.at[si, b], sem_arr.at[si]).wait()

        fire_gathers(0, 0)
        for t in range(n_groups):
            si = t % 2
            ni = 1 - si
            if t + 1 < n_groups:
                if t >= 1:
                    drain(wsem, ni, out_hbm.at[pl.ds(base, _CHUNK)])
                fire_gathers(t + 1, ni)
            drain(gsem, si, table_hbm.at[idx_v.at[pl.ds(0, _CHUNK)]])

            def scale_row(r, carry):
                for b in range(_NBUF):
                    for q in range(_D // _LANES):
                        sl = pl.ds(q * _LANES, _LANES)
                        rows_v[si, b, r, sl] = rows_v[si, b, r, sl] * _SCALE
                return carry

            lax.fori_loop(0, _CHUNK, scale_row, None)

            for b in range(_NBUF):
                off = (t * _NBUF + b) * _CHUNK
                pltpu.async_copy(
                    rows_v.at[si, b],
                    out_hbm.at[pl.ds(base + off, _CHUNK)],
                    wsem.at[si],
                )
        drain(wsem, (n_groups - 1) % 2, out_hbm.at[pl.ds(base, _CHUNK)])

    return k


def kernel(x, embedding):
    b0, s = x.shape
    batch = b0 * s
    xf = x.reshape(batch)
    info = plsc.get_sparse_core_info()
    out = _make(embedding.shape[0], batch, info.num_cores, info.num_subcores)(
        xf, embedding
    )
    return out.reshape(b0, s, _D)
